# Initial kernel scaffold; baseline (speedup 1.0000x reference)
import jax
import jax.numpy as jnp
from jax.experimental import pallas as pl


def kernel(edge_length, edge_index, atom_type, per_atom_energy, per_edge_scales):
    raise NotImplementedError("write your pallas kernel here")


# speedup vs baseline: 391.6862x; 391.6862x over previous
import jax
import jax.numpy as jnp
from jax import lax
from jax.experimental import pallas as pl
from jax.experimental.pallas import tpu as pltpu
from jax.experimental.pallas import tpu_sc as plsc

N_NODES = 100000
NUM_TYPES = 16
R_MAX = 5.0

NC = 2
NS = 16
L = 16
NW = NC * NS
ROW = 128
ROWS_PER_W = 1568
CH_ROWS = 32
N_CHUNKS = ROWS_PER_W // CH_ROWS
CH_EDGES = CH_ROWS * ROW
E_PAD = NW * ROWS_PER_W * ROW
ACC_SLICE = 6272
Z_ROWS = ACC_SLICE // ROW
N_PAD = NS * ACC_SLICE


def _sc_body(len2d, ctr2d, nbr2d, spec_h, t13_h, out_h,
             acc_s, spec_v, t13_v, len_v, ctr_v, nbr_v, eng_v, sem):
    c = lax.axis_index("c")
    s = lax.axis_index("s")
    wid = s * NC + c
    base_row = wid * ROWS_PER_W

    pltpu.sync_copy(spec_h, spec_v)
    pltpu.sync_copy(t13_h, t13_v)

    zero16 = jnp.zeros((L,), jnp.float32)

    for j in range(ROW // L):
        eng_v[0, pl.ds(j * L, L)] = zero16

    def zrow(r, _):
        pltpu.sync_copy(eng_v.at[0],
                        acc_s.at[pl.ds(s * ACC_SLICE + r * ROW, ROW)])
        return 0

    lax.fori_loop(0, Z_ROWS, zrow, 0)
    plsc.subcore_barrier()

    inv_rmax = jnp.float32(1.0 / R_MAX)

    def chunk_body(k, _):
        r0 = base_row + k * CH_ROWS
        pltpu.sync_copy(len2d.at[pl.ds(r0, CH_ROWS)], len_v)
        pltpu.sync_copy(ctr2d.at[pl.ds(r0, CH_ROWS)], ctr_v)
        pltpu.sync_copy(nbr2d.at[pl.ds(r0, CH_ROWS)], nbr_v)

        def row_body(r, _):
            for j in range(ROW // L):
                sl = pl.ds(j * L, L)
                ln = len_v[r, sl]
                ci = ctr_v[r, sl]
                ni = nbr_v[r, sl]
                sp_c = plsc.load_gather(spec_v, [ci])
                sp_n = plsc.load_gather(spec_v, [ni])
                t13 = plsc.load_gather(t13_v, [sp_c * NUM_TYPES + sp_n])
                inv = 1.0 / ln
                i2 = inv * inv
                i4 = i2 * i2
                i8 = i4 * i4
                i12 = i8 * i4
                rr = ln * inv_rmax
                r2 = rr * rr
                r6 = r2 * r2 * r2
                poly = 1.0 + r6 * (-28.0 + rr * (48.0 - 21.0 * rr))
                cut = jnp.where(rr < 1.0, poly, jnp.float32(0.0))
                eng_v[r, sl] = i12 * t13 * cut
            return 0

        lax.fori_loop(0, CH_ROWS, row_body, 0)

        def fire(r, _):
            pltpu.async_copy(eng_v.at[r], acc_s.at[ctr_v.at[r]], sem,
                             add=True)
            return 0

        lax.fori_loop(0, CH_ROWS, fire, 0)

        def drain(r, _):
            pltpu.make_async_copy(eng_v.at[r], acc_s.at[ctr_v.at[r]],
                                  sem).wait()
            return 0

        lax.fori_loop(0, CH_ROWS, drain, 0)
        return 0

    lax.fori_loop(0, N_CHUNKS, chunk_body, 0)
    plsc.subcore_barrier()

    pltpu.sync_copy(acc_s.at[pl.ds(s * ACC_SLICE, ACC_SLICE)],
                    out_h.at[pl.ds(c * N_PAD + s * ACC_SLICE, ACC_SLICE)])


def _combine_body(pae_ref, p_ref, o_ref):
    o_ref[...] = pae_ref[...] + p_ref[0] + p_ref[1]


@jax.jit
def _impl(edge_length, edge_index, atom_type, per_atom_energy, per_edge_scales):
    E = edge_length.shape[0]
    pad = E_PAD - E
    species = atom_type[:, 0].astype(jnp.int32)
    ctr = edge_index[0].astype(jnp.int32)
    nbr = edge_index[1].astype(jnp.int32)
    len_p = jnp.concatenate(
        [edge_length, jnp.full((pad,), 2.0 * R_MAX, jnp.float32)]
    ).reshape(-1, ROW)
    ctr_p = jnp.concatenate([ctr, jnp.zeros((pad,), jnp.int32)]).reshape(-1, ROW)
    nbr_p = jnp.concatenate([nbr, jnp.zeros((pad,), jnp.int32)]).reshape(-1, ROW)
    t13 = (per_edge_scales.astype(jnp.float32) ** 13 / 24.0).reshape(-1)

    mesh = plsc.VectorSubcoreMesh(core_axis_name="c", subcore_axis_name="s")
    partial = pl.kernel(
        _sc_body,
        out_type=jax.ShapeDtypeStruct((NC * N_PAD,), jnp.float32),
        mesh=mesh,
        compiler_params=pltpu.CompilerParams(needs_layout_passes=False),
        scratch_types=[
            pltpu.VMEM_SHARED((N_PAD,), jnp.float32),
            pltpu.VMEM((N_NODES,), jnp.int32),
            pltpu.VMEM((NUM_TYPES * NUM_TYPES,), jnp.float32),
            pltpu.VMEM((CH_ROWS, ROW), jnp.float32),
            pltpu.VMEM((CH_ROWS, ROW), jnp.int32),
            pltpu.VMEM((CH_ROWS, ROW), jnp.int32),
            pltpu.VMEM((CH_ROWS, ROW), jnp.float32),
            pltpu.SemaphoreType.DMA,
        ],
    )(len_p, ctr_p, nbr_p, species, t13)

    pae = per_atom_energy[:, 0]
    pae_p = jnp.concatenate(
        [pae, jnp.zeros((N_PAD - N_NODES,), jnp.float32)]
    ).reshape(-1, ROW)
    out = pl.pallas_call(
        _combine_body,
        out_shape=jax.ShapeDtypeStruct((N_PAD // ROW, ROW), jnp.float32),
    )(pae_p, partial.reshape(NC, N_PAD // ROW, ROW))
    return out.reshape(-1)[:N_NODES, None]


def kernel(edge_length, edge_index, atom_type, per_atom_energy, per_edge_scales):
    return _impl(edge_length, edge_index, atom_type, per_atom_energy,
                 per_edge_scales)


# speedup vs baseline: 519.4292x; 1.3261x over previous
import jax
import jax.numpy as jnp
from jax import lax
from jax.experimental import pallas as pl
from jax.experimental.pallas import tpu as pltpu
from jax.experimental.pallas import tpu_sc as plsc

N_NODES = 100000
NUM_TYPES = 16
R_MAX = 5.0

NC = 2
NS = 16
L = 16
NW = NC * NS
ROW = 128
CH_ROWS = 16
E_ROWS = 50000
TOTAL_CHUNKS = E_ROWS // CH_ROWS
KMAX = -(-TOTAL_CHUNKS // NW)
ACC_SLICE = 6272
Z_ROWS = ACC_SLICE // ROW
N_PAD = NS * ACC_SLICE


def _sc_body(len2d, ei3d, spec_h, t13_h, out_h,
             acc_s, spec_v, t13_v, len_v, ctr_v, nbr_v, eng_v, lsem, ssem):
    c = lax.axis_index("c")
    s = lax.axis_index("s")
    wid = s * NC + c

    pltpu.sync_copy(spec_h, spec_v)
    pltpu.sync_copy(t13_h, t13_v)

    zero16 = jnp.zeros((L,), jnp.float32)
    for j in range(ROW // L):
        eng_v[0, 0, pl.ds(j * L, L)] = zero16

    def zrow(r, _):
        pltpu.sync_copy(eng_v.at[0, 0],
                        acc_s.at[pl.ds(s * ACC_SLICE + r * ROW, ROW)])
        return 0

    lax.fori_loop(0, Z_ROWS, zrow, 0)
    plsc.subcore_barrier()

    inv_rmax = jnp.float32(1.0 / R_MAX)

    def fire_loads(k):
        cid = k * NW + wid
        b = lax.rem(k, 2)

        @pl.when(cid < TOTAL_CHUNKS)
        def _():
            r0 = cid * CH_ROWS
            pltpu.async_copy(len2d.at[pl.ds(r0, CH_ROWS)], len_v.at[b], lsem)
            pltpu.async_copy(ei3d.at[0, pl.ds(r0, CH_ROWS)], ctr_v.at[b], lsem)
            pltpu.async_copy(ei3d.at[1, pl.ds(r0, CH_ROWS)], nbr_v.at[b], lsem)

    def wait_loads(k):
        cid = k * NW + wid
        b = lax.rem(k, 2)

        @pl.when(cid < TOTAL_CHUNKS)
        def _():
            r0 = cid * CH_ROWS
            pltpu.make_async_copy(len2d.at[pl.ds(r0, CH_ROWS)], len_v.at[b],
                                  lsem).wait()
            pltpu.make_async_copy(ei3d.at[0, pl.ds(r0, CH_ROWS)],
                                  ctr_v.at[b], lsem).wait()
            pltpu.make_async_copy(ei3d.at[1, pl.ds(r0, CH_ROWS)],
                                  nbr_v.at[b], lsem).wait()

    def fire_scatter(k):
        cid = k * NW + wid
        b = lax.rem(k, 2)

        @pl.when(cid < TOTAL_CHUNKS)
        def _():
            def fire(r, _):
                pltpu.async_copy(eng_v.at[b, r], acc_s.at[ctr_v.at[b, r]],
                                 ssem, add=True)
                return 0

            lax.fori_loop(0, CH_ROWS, fire, 0)

    def drain_scatter(k):
        cid = k * NW + wid
        b = lax.rem(k, 2)

        @pl.when(jnp.logical_and(k >= 0, cid < TOTAL_CHUNKS))
        def _():
            def drain(r, _):
                pltpu.make_async_copy(eng_v.at[b, r],
                                      acc_s.at[ctr_v.at[b, r]], ssem).wait()
                return 0

            lax.fori_loop(0, CH_ROWS, drain, 0)

    fire_loads(0)

    def step(k, _):
        b = lax.rem(k, 2)
        cid = k * NW + wid
        drain_scatter(k - 1)
        fire_loads(k + 1)
        wait_loads(k)

        @pl.when(cid < TOTAL_CHUNKS)
        def _():
            def row_body(r, _):
                for j in range(ROW // L):
                    sl = pl.ds(j * L, L)
                    ln = len_v[b, r, sl]
                    ci = ctr_v[b, r, sl]
                    ni = nbr_v[b, r, sl]
                    sp_c = plsc.load_gather(spec_v, [ci])
                    sp_n = plsc.load_gather(spec_v, [ni])
                    t13 = plsc.load_gather(t13_v, [sp_c * NUM_TYPES + sp_n])
                    inv = 1.0 / ln
                    i2 = inv * inv
                    i4 = i2 * i2
                    i8 = i4 * i4
                    i12 = i8 * i4
                    rr = ln * inv_rmax
                    r2 = rr * rr
                    r6 = r2 * r2 * r2
                    poly = 1.0 + r6 * (-28.0 + rr * (48.0 - 21.0 * rr))
                    cut = jnp.where(rr < 1.0, poly, jnp.float32(0.0))
                    eng_v[b, r, sl] = i12 * t13 * cut
                return 0

            lax.fori_loop(0, CH_ROWS, row_body, 0)

        fire_scatter(k)
        return 0

    lax.fori_loop(0, KMAX, step, 0)
    drain_scatter(KMAX - 1)
    plsc.subcore_barrier()

    pltpu.sync_copy(acc_s.at[pl.ds(s * ACC_SLICE, ACC_SLICE)],
                    out_h.at[pl.ds(c * N_PAD + s * ACC_SLICE, ACC_SLICE)])


def _combine_body(pae_ref, p_ref, o_ref):
    o_ref[...] = pae_ref[...] + p_ref[0] + p_ref[1]


@jax.jit
def _impl(edge_length, edge_index, atom_type, per_atom_energy, per_edge_scales):
    species = atom_type[:, 0].astype(jnp.int32)
    len2d = edge_length.reshape(E_ROWS, ROW)
    ei3d = edge_index.astype(jnp.int32).reshape(2, E_ROWS, ROW)
    t13 = (per_edge_scales.astype(jnp.float32) ** 13 / 24.0).reshape(-1)

    mesh = plsc.VectorSubcoreMesh(core_axis_name="c", subcore_axis_name="s")
    partial = pl.kernel(
        _sc_body,
        out_type=jax.ShapeDtypeStruct((NC * N_PAD,), jnp.float32),
        mesh=mesh,
        compiler_params=pltpu.CompilerParams(needs_layout_passes=False),
        scratch_types=[
            pltpu.VMEM_SHARED((N_PAD,), jnp.float32),
            pltpu.VMEM((N_NODES,), jnp.int32),
            pltpu.VMEM((NUM_TYPES * NUM_TYPES,), jnp.float32),
            pltpu.VMEM((2, CH_ROWS, ROW), jnp.float32),
            pltpu.VMEM((2, CH_ROWS, ROW), jnp.int32),
            pltpu.VMEM((2, CH_ROWS, ROW), jnp.int32),
            pltpu.VMEM((2, CH_ROWS, ROW), jnp.float32),
            pltpu.SemaphoreType.DMA,
            pltpu.SemaphoreType.DMA,
        ],
    )(len2d, ei3d, species, t13)

    pae = per_atom_energy[:, 0]
    pae_p = jnp.concatenate(
        [pae, jnp.zeros((N_PAD - N_NODES,), jnp.float32)]
    ).reshape(-1, ROW)
    out = pl.pallas_call(
        _combine_body,
        out_shape=jax.ShapeDtypeStruct((N_PAD // ROW, ROW), jnp.float32),
    )(pae_p, partial.reshape(NC, N_PAD // ROW, ROW))
    return out.reshape(-1)[:N_NODES, None]


def kernel(edge_length, edge_index, atom_type, per_atom_energy, per_edge_scales):
    return _impl(edge_length, edge_index, atom_type, per_atom_energy,
                 per_edge_scales)


# speedup vs baseline: 520.6346x; 1.0023x over previous
import jax
import jax.numpy as jnp
from jax import lax
from jax.experimental import pallas as pl
from jax.experimental.pallas import tpu as pltpu
from jax.experimental.pallas import tpu_sc as plsc

N_NODES = 100000
NUM_TYPES = 16
R_MAX = 5.0

NC = 2
NS = 16
L = 16
NW = NC * NS
ROW = 128
CH_ROWS = 16
E_ROWS = 50000
TOTAL_CHUNKS = E_ROWS // CH_ROWS
KMAX = -(-TOTAL_CHUNKS // NW)
N_STRIDE = 100352
SPEC_WORDS = N_NODES // 8


def _sc_body(len2d, ei3d, spec4_h, t13_h, out_h,
             acc_v, spec4_v, t13_v, len_v, ctr_v, nbr_v, lsem):
    c = lax.axis_index("c")
    s = lax.axis_index("s")
    wid = s * NC + c

    pltpu.sync_copy(spec4_h, spec4_v)
    pltpu.sync_copy(t13_h, t13_v)

    zero16 = jnp.zeros((L,), jnp.float32)

    def zgrp(i, _):
        acc_v[pl.ds(i * L, L)] = zero16
        return 0

    lax.fori_loop(0, N_NODES // L, zgrp, 0)

    inv_rmax = jnp.float32(1.0 / R_MAX)

    def fire_loads(k):
        cid = k * NW + wid
        b = lax.rem(k, 2)

        @pl.when(cid < TOTAL_CHUNKS)
        def _():
            r0 = cid * CH_ROWS
            pltpu.async_copy(len2d.at[pl.ds(r0, CH_ROWS)], len_v.at[b], lsem)
            pltpu.async_copy(ei3d.at[0, pl.ds(r0, CH_ROWS)], ctr_v.at[b], lsem)
            pltpu.async_copy(ei3d.at[1, pl.ds(r0, CH_ROWS)], nbr_v.at[b], lsem)

    def wait_loads(k):
        cid = k * NW + wid
        b = lax.rem(k, 2)

        @pl.when(cid < TOTAL_CHUNKS)
        def _():
            r0 = cid * CH_ROWS
            pltpu.make_async_copy(len2d.at[pl.ds(r0, CH_ROWS)], len_v.at[b],
                                  lsem).wait()
            pltpu.make_async_copy(ei3d.at[0, pl.ds(r0, CH_ROWS)],
                                  ctr_v.at[b], lsem).wait()
            pltpu.make_async_copy(ei3d.at[1, pl.ds(r0, CH_ROWS)],
                                  nbr_v.at[b], lsem).wait()

    def unpack_species(word, idx):
        sh = (idx & 7) << 2
        return lax.shift_right_logical(word, sh) & 0xF

    fire_loads(0)

    def step(k, _):
        b = lax.rem(k, 2)
        cid = k * NW + wid
        fire_loads(k + 1)
        wait_loads(k)

        @pl.when(cid < TOTAL_CHUNKS)
        def _():
            def row_body(r, _):
                for j in range(ROW // L):
                    sl = pl.ds(j * L, L)
                    ln = len_v[b, r, sl]
                    ci = ctr_v[b, r, sl]
                    ni = nbr_v[b, r, sl]
                    wc = plsc.load_gather(spec4_v,
                                          [lax.shift_right_logical(ci, 3)])
                    wn = plsc.load_gather(spec4_v,
                                          [lax.shift_right_logical(ni, 3)])
                    sp_c = unpack_species(wc, ci)
                    sp_n = unpack_species(wn, ni)
                    t13 = plsc.load_gather(t13_v, [(sp_c << 4) | sp_n])
                    inv = 1.0 / ln
                    i2 = inv * inv
                    i4 = i2 * i2
                    i8 = i4 * i4
                    i12 = i8 * i4
                    rr = ln * inv_rmax
                    r2 = rr * rr
                    r6 = r2 * r2 * r2
                    poly = 1.0 + r6 * (-28.0 + rr * (48.0 - 21.0 * rr))
                    cut = jnp.where(rr < 1.0, poly, jnp.float32(0.0))
                    plsc.addupdate_scatter(acc_v, [ci], i12 * t13 * cut)
                return 0

            lax.fori_loop(0, CH_ROWS, row_body, 0)

        return 0

    lax.fori_loop(0, KMAX, step, 0)

    pltpu.sync_copy(acc_v, out_h.at[pl.ds(wid * N_STRIDE, N_NODES)])


def _combine_body(pae_ref, p_ref, o_ref):
    o_ref[...] = pae_ref[...] + jnp.sum(p_ref[...], axis=0)


@jax.jit
def _impl(edge_length, edge_index, atom_type, per_atom_energy, per_edge_scales):
    species = atom_type[:, 0].astype(jnp.uint32)
    shifts = (jnp.arange(8, dtype=jnp.uint32) * 4)[None, :]
    spec4 = lax.bitcast_convert_type(
        (species.reshape(SPEC_WORDS, 8) << shifts).sum(
            axis=1, dtype=jnp.uint32), jnp.int32)
    len2d = edge_length.reshape(E_ROWS, ROW)
    ei3d = edge_index.astype(jnp.int32).reshape(2, E_ROWS, ROW)
    t13 = (per_edge_scales.astype(jnp.float32) ** 13 / 24.0).reshape(-1)

    mesh = plsc.VectorSubcoreMesh(core_axis_name="c", subcore_axis_name="s")
    partial = pl.kernel(
        _sc_body,
        out_type=jax.ShapeDtypeStruct((NW * N_STRIDE,), jnp.float32),
        mesh=mesh,
        compiler_params=pltpu.CompilerParams(needs_layout_passes=False),
        scratch_types=[
            pltpu.VMEM((N_NODES,), jnp.float32),
            pltpu.VMEM((SPEC_WORDS,), jnp.int32),
            pltpu.VMEM((NUM_TYPES * NUM_TYPES,), jnp.float32),
            pltpu.VMEM((2, CH_ROWS, ROW), jnp.float32),
            pltpu.VMEM((2, CH_ROWS, ROW), jnp.int32),
            pltpu.VMEM((2, CH_ROWS, ROW), jnp.int32),
            pltpu.SemaphoreType.DMA,
        ],
    )(len2d, ei3d, spec4, t13)

    pae = per_atom_energy[:, 0]
    pae_p = jnp.concatenate(
        [pae, jnp.zeros((N_STRIDE - N_NODES,), jnp.float32)]
    ).reshape(-1, ROW)
    out = pl.pallas_call(
        _combine_body,
        out_shape=jax.ShapeDtypeStruct((N_STRIDE // ROW, ROW), jnp.float32),
    )(pae_p, partial.reshape(NW, N_STRIDE // ROW, ROW))
    return out.reshape(-1)[:N_NODES, None]


def kernel(edge_length, edge_index, atom_type, per_atom_energy, per_edge_scales):
    return _impl(edge_length, edge_index, atom_type, per_atom_energy,
                 per_edge_scales)


# speedup vs baseline: 1049.9202x; 2.0166x over previous
import jax
import jax.numpy as jnp
from jax import lax
from jax.experimental import pallas as pl
from jax.experimental.pallas import tpu as pltpu
from jax.experimental.pallas import tpu_sc as plsc

N_NODES = 100000
NUM_TYPES = 16
R_MAX = 5.0

NC = 2
NS = 16
L = 16
NW = NC * NS
ROW = 128
CH_ROWS = 16
E_ROWS = 50000
TOTAL_CHUNKS = E_ROWS // CH_ROWS
KMAX = -(-TOTAL_CHUNKS // NW)
N_STRIDE = 100352
SPEC_WORDS = N_NODES // 8


def _sc_body(len2d, ei3d, spec4_h, t13_h, out_h,
             acc_v, spec4_v, t13_v, len_v, ctr_v, nbr_v, lsem):
    c = lax.axis_index("c")
    s = lax.axis_index("s")
    wid = s * NC + c

    pltpu.sync_copy(spec4_h, spec4_v)
    pltpu.sync_copy(t13_h, t13_v)

    zero16 = jnp.zeros((L,), jnp.float32)

    @plsc.parallel_loop(0, N_NODES // L, unroll=4)
    def _zero(i):
        acc_v[pl.ds(i * L, L)] = zero16

    inv_rmax = jnp.float32(1.0 / R_MAX)

    def fire_loads(k):
        cid = k * NW + wid
        b = lax.rem(k, 2)

        @pl.when(cid < TOTAL_CHUNKS)
        def _():
            r0 = cid * CH_ROWS
            pltpu.async_copy(len2d.at[pl.ds(r0, CH_ROWS)], len_v.at[b], lsem)
            pltpu.async_copy(ei3d.at[0, pl.ds(r0, CH_ROWS)], ctr_v.at[b], lsem)
            pltpu.async_copy(ei3d.at[1, pl.ds(r0, CH_ROWS)], nbr_v.at[b], lsem)

    def wait_loads(k):
        cid = k * NW + wid
        b = lax.rem(k, 2)

        @pl.when(cid < TOTAL_CHUNKS)
        def _():
            r0 = cid * CH_ROWS
            pltpu.make_async_copy(len2d.at[pl.ds(r0, CH_ROWS)], len_v.at[b],
                                  lsem).wait()
            pltpu.make_async_copy(ei3d.at[0, pl.ds(r0, CH_ROWS)],
                                  ctr_v.at[b], lsem).wait()
            pltpu.make_async_copy(ei3d.at[1, pl.ds(r0, CH_ROWS)],
                                  nbr_v.at[b], lsem).wait()

    def unpack_species(word, idx):
        sh = (idx & 7) << 2
        return lax.shift_right_logical(word, sh) & 0xF

    fire_loads(0)

    def step(k, _):
        b = lax.rem(k, 2)
        cid = k * NW + wid
        fire_loads(k + 1)
        wait_loads(k)

        @pl.when(cid < TOTAL_CHUNKS)
        def _():
            @plsc.parallel_loop(0, CH_ROWS)
            def row_body(r):
                for j in range(ROW // L):
                    sl = pl.ds(j * L, L)
                    ln = len_v[b, r, sl]
                    ci = ctr_v[b, r, sl]
                    ni = nbr_v[b, r, sl]
                    wc = plsc.load_gather(spec4_v,
                                          [lax.shift_right_logical(ci, 3)])
                    wn = plsc.load_gather(spec4_v,
                                          [lax.shift_right_logical(ni, 3)])
                    sp_c = unpack_species(wc, ci)
                    sp_n = unpack_species(wn, ni)
                    t13 = plsc.load_gather(t13_v, [(sp_c << 4) | sp_n])
                    inv = 1.0 / ln
                    i2 = inv * inv
                    i4 = i2 * i2
                    i8 = i4 * i4
                    i12 = i8 * i4
                    rr = ln * inv_rmax
                    r2 = rr * rr
                    r6 = r2 * r2 * r2
                    poly = 1.0 + r6 * (-28.0 + rr * (48.0 - 21.0 * rr))
                    cut = jnp.where(rr < 1.0, poly, jnp.float32(0.0))
                    plsc.addupdate_scatter(acc_v, [ci], i12 * t13 * cut)

        return 0

    lax.fori_loop(0, KMAX, step, 0)

    pltpu.sync_copy(acc_v, out_h.at[pl.ds(wid * N_STRIDE, N_NODES)])


def _combine_body(pae_ref, p_ref, o_ref):
    o_ref[...] = pae_ref[...] + jnp.sum(p_ref[...], axis=0)


@jax.jit
def _impl(edge_length, edge_index, atom_type, per_atom_energy, per_edge_scales):
    species = atom_type[:, 0].astype(jnp.uint32)
    shifts = (jnp.arange(8, dtype=jnp.uint32) * 4)[None, :]
    spec4 = lax.bitcast_convert_type(
        (species.reshape(SPEC_WORDS, 8) << shifts).sum(
            axis=1, dtype=jnp.uint32), jnp.int32)
    len2d = edge_length.reshape(E_ROWS, ROW)
    ei3d = edge_index.astype(jnp.int32).reshape(2, E_ROWS, ROW)
    t13 = (per_edge_scales.astype(jnp.float32) ** 13 / 24.0).reshape(-1)

    mesh = plsc.VectorSubcoreMesh(core_axis_name="c", subcore_axis_name="s")
    partial = pl.kernel(
        _sc_body,
        out_type=jax.ShapeDtypeStruct((NW * N_STRIDE,), jnp.float32),
        mesh=mesh,
        compiler_params=pltpu.CompilerParams(needs_layout_passes=False),
        scratch_types=[
            pltpu.VMEM((N_NODES,), jnp.float32),
            pltpu.VMEM((SPEC_WORDS,), jnp.int32),
            pltpu.VMEM((NUM_TYPES * NUM_TYPES,), jnp.float32),
            pltpu.VMEM((2, CH_ROWS, ROW), jnp.float32),
            pltpu.VMEM((2, CH_ROWS, ROW), jnp.int32),
            pltpu.VMEM((2, CH_ROWS, ROW), jnp.int32),
            pltpu.SemaphoreType.DMA,
        ],
    )(len2d, ei3d, spec4, t13)

    pae = per_atom_energy[:, 0]
    pae_p = jnp.concatenate(
        [pae, jnp.zeros((N_STRIDE - N_NODES,), jnp.float32)]
    ).reshape(-1, ROW)
    out = pl.pallas_call(
        _combine_body,
        out_shape=jax.ShapeDtypeStruct((N_STRIDE // ROW, ROW), jnp.float32),
    )(pae_p, partial.reshape(NW, N_STRIDE // ROW, ROW))
    return out.reshape(-1)[:N_NODES, None]


def kernel(edge_length, edge_index, atom_type, per_atom_energy, per_edge_scales):
    return _impl(edge_length, edge_index, atom_type, per_atom_energy,
                 per_edge_scales)


# speedup vs baseline: 1222.3551x; 1.1642x over previous
import jax
import jax.numpy as jnp
from jax import lax
from jax.experimental import pallas as pl
from jax.experimental.pallas import tpu as pltpu
from jax.experimental.pallas import tpu_sc as plsc

N_NODES = 100000
NUM_TYPES = 16
R_MAX = 5.0

NC = 2
NS = 16
L = 16
NW = NC * NS
N_EDGES = 6400000
CH_EDGES = 2048
N_GROUPS = CH_EDGES // L
TOTAL_CHUNKS = N_EDGES // CH_EDGES
KMAX = -(-TOTAL_CHUNKS // NW)
N_STRIDE = 100352
SPEC_WORDS = N_NODES // 8


def _sc_body(len_h, ei_h, spec4_h, t13_h, out_h,
             acc_v, spec4_v, t13_v, len_v, ctr_v, nbr_v, lsem):
    c = lax.axis_index("c")
    s = lax.axis_index("s")
    wid = s * NC + c

    pltpu.sync_copy(spec4_h, spec4_v)
    pltpu.sync_copy(t13_h, t13_v)

    zero16 = jnp.zeros((L,), jnp.float32)

    @plsc.parallel_loop(0, N_NODES // L, unroll=4)
    def _zero(i):
        acc_v[pl.ds(i * L, L)] = zero16

    inv_rmax = jnp.float32(1.0 / R_MAX)

    def fire_loads(k):
        cid = k * NW + wid
        b = lax.rem(k, 2)

        @pl.when(cid < TOTAL_CHUNKS)
        def _():
            e0 = cid * CH_EDGES
            pltpu.async_copy(len_h.at[pl.ds(e0, CH_EDGES)], len_v.at[b], lsem)
            pltpu.async_copy(ei_h.at[0, pl.ds(e0, CH_EDGES)], ctr_v.at[b], lsem)
            pltpu.async_copy(ei_h.at[1, pl.ds(e0, CH_EDGES)], nbr_v.at[b], lsem)

    def wait_loads(k):
        cid = k * NW + wid
        b = lax.rem(k, 2)

        @pl.when(cid < TOTAL_CHUNKS)
        def _():
            e0 = cid * CH_EDGES
            pltpu.make_async_copy(len_h.at[pl.ds(e0, CH_EDGES)], len_v.at[b],
                                  lsem).wait()
            pltpu.make_async_copy(ei_h.at[0, pl.ds(e0, CH_EDGES)],
                                  ctr_v.at[b], lsem).wait()
            pltpu.make_async_copy(ei_h.at[1, pl.ds(e0, CH_EDGES)],
                                  nbr_v.at[b], lsem).wait()

    def unpack_species(word, idx):
        sh = (idx & 7) << 2
        return lax.shift_right_logical(word, sh) & 0xF

    fire_loads(0)

    def step(k, _):
        b = lax.rem(k, 2)
        cid = k * NW + wid
        fire_loads(k + 1)
        wait_loads(k)

        @pl.when(cid < TOTAL_CHUNKS)
        def _():
            @plsc.parallel_loop(0, N_GROUPS)
            def _grp(g):
                sl = pl.ds(g * L, L)
                ln = len_v[b, sl]
                ci = ctr_v[b, sl]
                ni = nbr_v[b, sl]
                wc = plsc.load_gather(spec4_v,
                                      [lax.shift_right_logical(ci, 3)])
                wn = plsc.load_gather(spec4_v,
                                      [lax.shift_right_logical(ni, 3)])
                sp_c = unpack_species(wc, ci)
                sp_n = unpack_species(wn, ni)
                t13 = plsc.load_gather(t13_v, [(sp_c << 4) | sp_n])
                inv = 1.0 / ln
                i2 = inv * inv
                i4 = i2 * i2
                i8 = i4 * i4
                i12 = i8 * i4
                rr = ln * inv_rmax
                r2 = rr * rr
                r6 = r2 * r2 * r2
                poly = 1.0 + r6 * (-28.0 + rr * (48.0 - 21.0 * rr))
                cut = jnp.where(rr < 1.0, poly, jnp.float32(0.0))
                plsc.addupdate_scatter(acc_v, [ci], i12 * t13 * cut)

        return 0

    lax.fori_loop(0, KMAX, step, 0)

    pltpu.sync_copy(acc_v, out_h.at[pl.ds(wid * N_STRIDE, N_NODES)])


def _combine_body(pae_ref, p_ref, o_ref):
    acc = pae_ref[...]
    for w in range(NW):
        acc = acc + p_ref[pl.ds(w * N_STRIDE, N_NODES)]
    o_ref[...] = acc


@jax.jit
def _impl(edge_length, edge_index, atom_type, per_atom_energy, per_edge_scales):
    species = atom_type[:, 0].astype(jnp.uint32)
    shifts = (jnp.arange(8, dtype=jnp.uint32) * 4)[None, :]
    spec4 = lax.bitcast_convert_type(
        (species.reshape(SPEC_WORDS, 8) << shifts).sum(
            axis=1, dtype=jnp.uint32), jnp.int32)
    ei = edge_index.astype(jnp.int32)
    t13 = (per_edge_scales.astype(jnp.float32) ** 13 / 24.0).reshape(-1)

    mesh = plsc.VectorSubcoreMesh(core_axis_name="c", subcore_axis_name="s")
    partial = pl.kernel(
        _sc_body,
        out_type=jax.ShapeDtypeStruct((NW * N_STRIDE,), jnp.float32),
        mesh=mesh,
        compiler_params=pltpu.CompilerParams(needs_layout_passes=False),
        scratch_types=[
            pltpu.VMEM((N_NODES,), jnp.float32),
            pltpu.VMEM((SPEC_WORDS,), jnp.int32),
            pltpu.VMEM((NUM_TYPES * NUM_TYPES,), jnp.float32),
            pltpu.VMEM((2, CH_EDGES), jnp.float32),
            pltpu.VMEM((2, CH_EDGES), jnp.int32),
            pltpu.VMEM((2, CH_EDGES), jnp.int32),
            pltpu.SemaphoreType.DMA,
        ],
    )(edge_length, ei, spec4, t13)

    pae = per_atom_energy[:, 0]
    out = pl.pallas_call(
        _combine_body,
        out_shape=jax.ShapeDtypeStruct((N_NODES,), jnp.float32),
    )(pae, partial)
    return out[:, None]


def kernel(edge_length, edge_index, atom_type, per_atom_energy, per_edge_scales):
    return _impl(edge_length, edge_index, atom_type, per_atom_energy,
                 per_edge_scales)
